# baseline (speedup 1.0000x reference)
import functools

import jax
import jax.numpy as jnp
from jax import lax
from jax.experimental import pallas as pl
from jax.experimental.pallas import tpu as pltpu
from jax.experimental.pallas import tpu_sc as plsc

N = 10000
E = 160000
D = 256
H = 128
NC = 2
NS = 16
C = 64
NB = 4
EPT = 10240
E2 = NS * EPT
NCHUNK = EPT // C
NG = NCHUNK // NB
RC = 80
NRCH = N // RC


def _sc_body(xcat, src2, dstp, z_agg, z_deg,
             agg, degp,
             idx_s0, idx_s1, idx_s2, idx_s3,
             idx_d0, idx_d1, idx_d2, idx_d3,
             rows0, rows1, rows2, rows3, deg_local,
             agg_sp,
             sg0, sg1, sg2, sg3, si0, si1, si2, si3):
    idx_s = [idx_s0, idx_s1, idx_s2, idx_s3]
    idx_d = [idx_d0, idx_d1, idx_d2, idx_d3]
    rows = [rows0, rows1, rows2, rows3]
    sem_g = [sg0, sg1, sg2, sg3]
    sem_i = [si0, si1, si2, si3]
    c = lax.axis_index("c")
    s = lax.axis_index("s")

    n_i = jnp.where(s < NRCH - (NRCH // NS) * NS, NRCH // NS + 1, NRCH // NS)

    def over_row_chunks(fn):
        def body(i, carry):
            fn(pl.ds(pl.multiple_of((s + NS * i) * RC, 8), RC))
            return carry

        lax.fori_loop(0, n_i, body, jnp.int32(0))

    zstage = rows0.at[pl.ds(0, RC)]
    pltpu.sync_copy(z_agg, zstage)
    pltpu.sync_copy(z_deg, deg_local)

    def zero_init(rs):
        pltpu.sync_copy(zstage, agg_sp.at[rs])

    over_row_chunks(zero_init)
    plsc.subcore_barrier()

    ones16 = jnp.ones((16,), jnp.float32)

    def idx_slices(chunk):
        base2 = pl.multiple_of(c * E2 + s * EPT + chunk * C, 8)
        based = pl.multiple_of(s * EPT + chunk * C, 8)
        return src2.at[pl.ds(base2, C)], dstp.at[pl.ds(based, C)]

    def fire_idx(chunk, k):
        ssrc, sdst = idx_slices(chunk)
        pltpu.async_copy(ssrc, idx_s[k], sem_i[k])
        pltpu.async_copy(sdst, idx_d[k], sem_i[k])

    def drain_idx(chunk, k):
        ssrc, sdst = idx_slices(chunk)
        pltpu.make_async_copy(ssrc, idx_s[k], sem_i[k]).wait()
        pltpu.make_async_copy(sdst, idx_d[k], sem_i[k]).wait()

    def fire_gather(k):
        pltpu.async_copy(xcat.at[idx_s[k]], rows[k], sem_g[k])

    def drain_gather(k):
        pltpu.make_async_copy(xcat.at[idx_s[k]], rows[k], sem_g[k]).wait()

    def process(k):
        pltpu.sync_copy(rows[k], agg_sp.at[idx_d[k]], add=True)
        for j in range(C // 16):
            plsc.addupdate_scatter(deg_local, [idx_d[k][pl.ds(j * 16, 16)]],
                                   ones16)

    for k in range(NB):
        ssrc0, sdst0 = idx_slices(k)
        pltpu.sync_copy(ssrc0, idx_s[k])
        pltpu.sync_copy(sdst0, idx_d[k])
        fire_gather(k)

    def group(g, carry):
        for k in range(NB):
            drain_gather(k)
            process(k)

            @pl.when(g < NG - 1)
            def _(k=k):
                fire_idx(NB * g + k + NB, k)

        @pl.when(g < NG - 1)
        def _():
            for k in range(NB):
                drain_idx(NB * g + k + NB, k)
                fire_gather(k)

        return carry

    lax.fori_loop(0, NG, group, jnp.int32(0))
    plsc.subcore_barrier()

    pltpu.sync_copy(deg_local, degp.at[c, s])

    def copy_out(rs):
        pltpu.sync_copy(agg_sp.at[rs], zstage)
        pltpu.sync_copy(zstage, agg.at[c, rs])

    over_row_chunks(copy_out)


def _sc_aggregate(x, src, dst):
    xcat = jnp.concatenate([x[:, :H], x[:, H:]], axis=0)
    pad = E2 - E
    srcp = jnp.concatenate([src, jnp.zeros((pad,), jnp.int32)])
    src2 = jnp.concatenate([srcp, srcp + N])
    dstp = jnp.concatenate([dst, jnp.full((pad,), N, jnp.int32)])
    z_agg = jnp.zeros((RC, H), jnp.float32)
    z_deg = jnp.zeros((N + 16,), jnp.float32)

    mesh = plsc.VectorSubcoreMesh(core_axis_name="c", subcore_axis_name="s")
    f = pl.kernel(
        _sc_body,
        out_type=(
            jax.ShapeDtypeStruct((NC, N, H), jnp.float32),
            jax.ShapeDtypeStruct((NC, NS, N + 16), jnp.float32),
        ),
        mesh=mesh,
        compiler_params=pltpu.CompilerParams(needs_layout_passes=False),
        scratch_types=(
            [pltpu.VMEM((C,), jnp.int32)] * (2 * NB)
            + [pltpu.VMEM((C, H), jnp.float32)] * NB
            + [
                pltpu.VMEM((N + 16,), jnp.float32),
                pltpu.VMEM_SHARED((N + 8, H), jnp.float32),
            ]
            + [pltpu.SemaphoreType.DMA] * (2 * NB)
        ),
        name="sage_sc_aggregate",
    )
    return f(xcat, src2, dstp, z_agg, z_deg)


R = 1000


def _tc_body(x, aa, ab, dp, Wl, bl, Wr, Wa, ba, W1, b1, W2, b2, W3p, b3p,
             out):
    deg = jnp.sum(dp[...], axis=1, keepdims=True) * 0.5
    inv = 1.0 / jnp.maximum(deg, 1.0)
    mean = jnp.concatenate([aa[...] * inv, ab[...] * inv], axis=1)
    h = (jnp.dot(mean, Wl[...], preferred_element_type=jnp.float32)
         + jnp.dot(x[...], Wr[...], preferred_element_type=jnp.float32)
         + bl[...])
    h = jnp.maximum(h, 0.0)
    h = jnp.maximum(jnp.dot(h, Wa[...], preferred_element_type=jnp.float32)
                    + ba[...], 0.0)
    h = jnp.maximum(jnp.dot(h, W1[...], preferred_element_type=jnp.float32)
                    + b1[...], 0.0)
    h = jnp.maximum(jnp.dot(h, W2[...], preferred_element_type=jnp.float32)
                    + b2[...], 0.0)
    out[...] = (jnp.dot(h, W3p[...], preferred_element_type=jnp.float32)
                + b3p[...])


def _tc_dense(x, aa, ab, degt, Wl, bl, Wr, Wa, ba, W1, b1, W2, b2, W3, b3):
    W3p = jnp.pad(W3, ((0, 0), (0, 125)))
    b3p = jnp.pad(b3, (0, 125))
    nblk = N // R

    def row_spec(cols):
        return pl.BlockSpec((R, cols), lambda i: (i, 0))

    def full_spec(arr):
        nd = arr.ndim
        return pl.BlockSpec(arr.shape, (lambda n: (lambda i: (0,) * n))(nd))

    weights = (Wl, bl, Wr, Wa, ba, W1, b1, W2, b2, W3p, b3p)
    grid_spec = pl.GridSpec(
        grid=(nblk,),
        in_specs=[row_spec(D), row_spec(H), row_spec(H),
                  row_spec(NC * NS)] + [full_spec(w) for w in weights],
        out_specs=row_spec(H),
    )
    return pl.pallas_call(
        _tc_body,
        grid_spec=grid_spec,
        out_shape=jax.ShapeDtypeStruct((N, H), jnp.float32),
    )(x, aa, ab, degt, *weights)


@jax.jit
def kernel(x, edge_index, W_l, b_l, W_r, W_a, b_a, W_1, b_1, W_2, b_2, W_3,
           b_3):
    src = edge_index[0]
    dst = edge_index[1]
    agg, degp = _sc_aggregate(x, src, dst)
    degt = degp.reshape(NC * NS, N + 16)[:, :N].T
    out = _tc_dense(x, agg[0], agg[1], degt, W_l, b_l, W_r, W_a,
                    b_a, W_1, b_1, W_2, b_2, W_3, b_3)
    return out[:, :3]


# speedup vs baseline: 1.7378x; 1.7378x over previous
import functools

import jax
import jax.numpy as jnp
from jax import lax
from jax.experimental import pallas as pl
from jax.experimental.pallas import tpu as pltpu
from jax.experimental.pallas import tpu_sc as plsc

N = 10000
E = 160000
D = 256
H = 128
NC = 2
NS = 16
C = 88
NB = 3
EPT = 10032
E2 = NS * EPT
NCHUNK = EPT // C
NG = NCHUNK // NB
RC = 80
NRCH = N // RC


def _sc_body(xcat, src2, dstp, z_agg, z_deg,
             agg, degp,
             idx_s0, idx_s1, idx_s2,
             idx_d0, idx_d1, idx_d2,
             rows0, rows1, rows2, deg_local,
             agg_sp,
             sg0, sg1, sg2, si0, si1, si2):
    idx_s = [idx_s0, idx_s1, idx_s2]
    idx_d = [idx_d0, idx_d1, idx_d2]
    rows = [rows0, rows1, rows2]
    sem_g = [sg0, sg1, sg2]
    sem_i = [si0, si1, si2]
    c = lax.axis_index("c")
    s = lax.axis_index("s")

    n_i = jnp.where(s < NRCH - (NRCH // NS) * NS, NRCH // NS + 1, NRCH // NS)

    def over_row_chunks(fn):
        def body(i, carry):
            fn(pl.ds(pl.multiple_of((s + NS * i) * RC, 8), RC))
            return carry

        lax.fori_loop(0, n_i, body, jnp.int32(0))

    zstage = rows0.at[pl.ds(0, RC)]
    pltpu.sync_copy(z_agg, zstage)
    pltpu.sync_copy(z_deg, deg_local)

    def zero_init(rs):
        pltpu.sync_copy(zstage, agg_sp.at[rs])

    over_row_chunks(zero_init)
    plsc.subcore_barrier()

    ones16 = jnp.ones((16,), jnp.float32)

    def idx_slices(chunk):
        base2 = pl.multiple_of(c * E2 + s * EPT + chunk * C, 8)
        based = pl.multiple_of(s * EPT + chunk * C, 8)
        return src2.at[pl.ds(base2, C)], dstp.at[pl.ds(based, C)]

    def fire_idx(chunk, k):
        ssrc, sdst = idx_slices(chunk)
        pltpu.async_copy(ssrc, idx_s[k], sem_i[k])
        pltpu.async_copy(sdst, idx_d[k], sem_i[k])

    def drain_idx(chunk, k):
        ssrc, sdst = idx_slices(chunk)
        pltpu.make_async_copy(ssrc, idx_s[k], sem_i[k]).wait()
        pltpu.make_async_copy(sdst, idx_d[k], sem_i[k]).wait()

    def fire_gather(k):
        pltpu.async_copy(xcat.at[idx_s[k]], rows[k], sem_g[k])

    def drain_gather(k):
        pltpu.make_async_copy(xcat.at[idx_s[k]], rows[k], sem_g[k]).wait()

    def process(k):
        pltpu.sync_copy(rows[k], agg_sp.at[idx_d[k]], add=True)
        for j in range(C // 16):
            plsc.addupdate_scatter(deg_local, [idx_d[k][pl.ds(j * 16, 16)]],
                                   ones16)

    for k in range(NB):
        ssrc0, sdst0 = idx_slices(k)
        pltpu.sync_copy(ssrc0, idx_s[k])
        pltpu.sync_copy(sdst0, idx_d[k])
        fire_gather(k)

    def group(g, carry):
        for k in range(NB):
            drain_gather(k)
            process(k)

            @pl.when(g < NG - 1)
            def _(k=k):
                fire_idx(NB * g + k + NB, k)

        @pl.when(g < NG - 1)
        def _():
            for k in range(NB):
                drain_idx(NB * g + k + NB, k)
                fire_gather(k)

        return carry

    lax.fori_loop(0, NG, group, jnp.int32(0))
    plsc.subcore_barrier()

    pltpu.sync_copy(deg_local, degp.at[c, s])

    def copy_out(rs):
        pltpu.sync_copy(agg_sp.at[rs], zstage)
        pltpu.sync_copy(zstage, agg.at[c, rs])

    over_row_chunks(copy_out)


def _sc_aggregate(x, src, dst):
    xcat = jnp.concatenate([x[:, :H], x[:, H:]], axis=0)
    pad = E2 - E
    srcp = jnp.concatenate([src, jnp.zeros((pad,), jnp.int32)])
    src2 = jnp.concatenate([srcp, srcp + N])
    dstp = jnp.concatenate([dst, jnp.full((pad,), N, jnp.int32)])
    z_agg = jnp.zeros((RC, H), jnp.float32)
    z_deg = jnp.zeros((N + 16,), jnp.float32)

    mesh = plsc.VectorSubcoreMesh(core_axis_name="c", subcore_axis_name="s")
    f = pl.kernel(
        _sc_body,
        out_type=(
            jax.ShapeDtypeStruct((NC, N, H), jnp.float32),
            jax.ShapeDtypeStruct((NC, NS, N + 16), jnp.float32),
        ),
        mesh=mesh,
        compiler_params=pltpu.CompilerParams(needs_layout_passes=False),
        scratch_types=(
            [pltpu.VMEM((C,), jnp.int32)] * (2 * NB)
            + [pltpu.VMEM((C, H), jnp.float32)] * NB
            + [
                pltpu.VMEM((N + 16,), jnp.float32),
                pltpu.VMEM_SHARED((N + 8, H), jnp.float32),
            ]
            + [pltpu.SemaphoreType.DMA] * (2 * NB)
        ),
        name="sage_sc_aggregate",
    )
    return f(xcat, src2, dstp, z_agg, z_deg)


R = 1000


def _tc_body(x, aa, ab, dp, Wl, bl, Wr, Wa, ba, W1, b1, W2, b2, W3p, b3p,
             out):
    deg = jnp.sum(dp[...], axis=1, keepdims=True) * 0.5
    inv = 1.0 / jnp.maximum(deg, 1.0)
    mean = jnp.concatenate([aa[...] * inv, ab[...] * inv], axis=1)
    h = (jnp.dot(mean, Wl[...], preferred_element_type=jnp.float32)
         + jnp.dot(x[...], Wr[...], preferred_element_type=jnp.float32)
         + bl[...])
    h = jnp.maximum(h, 0.0)
    h = jnp.maximum(jnp.dot(h, Wa[...], preferred_element_type=jnp.float32)
                    + ba[...], 0.0)
    h = jnp.maximum(jnp.dot(h, W1[...], preferred_element_type=jnp.float32)
                    + b1[...], 0.0)
    h = jnp.maximum(jnp.dot(h, W2[...], preferred_element_type=jnp.float32)
                    + b2[...], 0.0)
    out[...] = (jnp.dot(h, W3p[...], preferred_element_type=jnp.float32)
                + b3p[...])


def _tc_dense(x, aa, ab, degt, Wl, bl, Wr, Wa, ba, W1, b1, W2, b2, W3, b3):
    W3p = jnp.pad(W3, ((0, 0), (0, 125)))
    b3p = jnp.pad(b3, (0, 125))
    nblk = N // R

    def row_spec(cols):
        return pl.BlockSpec((R, cols), lambda i: (i, 0))

    def full_spec(arr):
        nd = arr.ndim
        return pl.BlockSpec(arr.shape, (lambda n: (lambda i: (0,) * n))(nd))

    weights = (Wl, bl, Wr, Wa, ba, W1, b1, W2, b2, W3p, b3p)
    grid_spec = pl.GridSpec(
        grid=(nblk,),
        in_specs=[row_spec(D), row_spec(H), row_spec(H),
                  row_spec(NC * NS)] + [full_spec(w) for w in weights],
        out_specs=row_spec(H),
    )
    return pl.pallas_call(
        _tc_body,
        grid_spec=grid_spec,
        out_shape=jax.ShapeDtypeStruct((N, H), jnp.float32),
    )(x, aa, ab, degt, *weights)


@jax.jit
def kernel(x, edge_index, W_l, b_l, W_r, W_a, b_a, W_1, b_1, W_2, b_2, W_3,
           b_3):
    src = edge_index[0]
    dst = edge_index[1]
    agg, degp = _sc_aggregate(x, src, dst)
    degt = degp.reshape(NC * NS, N + 16)[:, :N].T
    out = _tc_dense(x, agg[0], agg[1], degt, W_l, b_l, W_r, W_a,
                    b_a, W_1, b_1, W_2, b_2, W_3, b_3)
    return out[:, :3]
